# baseline (speedup 1.0000x reference)
import jax
import jax.numpy as jnp
from jax.experimental import pallas as pl
from jax.experimental.pallas import tpu as pltpu

_ALPHA = 0.2


def _lrelu(v):
    return jnp.maximum(v, _ALPHA * v)


def _mp_kernel(x_ref, w0a_ref, w0b_ref, w0d_ref, b0_ref, w1_ref, b1_ref,
               nw0_ref, nb0_ref, nw1_ref, nb1_ref, out_ref):
    x = x_ref[0]
    n = x.shape[0]

    u = jnp.dot(x, w0a_ref[...], preferred_element_type=jnp.float32)
    v = jnp.dot(x, w0b_ref[...], preferred_element_type=jnp.float32)
    v = v + b0_ref[...]

    xx = x * x
    g = jax.lax.dot_general(x, x, (((1,), (1,)), ((), ())),
                            preferred_element_type=jnp.float32)
    sq_col = jnp.sum(xx, axis=1, keepdims=True)
    ones_row = jnp.ones((1, x.shape[1]), jnp.float32)
    sq_row = jax.lax.dot_general(ones_row, xx, (((1,), (1,)), ((), ())),
                                 preferred_element_type=jnp.float32)
    dist = jnp.sqrt(jnp.maximum(sq_col + sq_row - 2.0 * g, 0.0))

    e = u[:, None, :] + v[None, :, :] + dist[:, :, None] * w0d_ref[...][None]
    e = _lrelu(e)

    e2 = jnp.dot(e.reshape(n * n, -1), w1_ref[...],
                 preferred_element_type=jnp.float32) + b1_ref[...]
    e2 = _lrelu(e2).reshape(n, n, -1)

    agg = jnp.sum(e2, axis=1)

    h = jnp.concatenate([agg, x], axis=1)
    h = _lrelu(jnp.dot(h, nw0_ref[...],
                       preferred_element_type=jnp.float32) + nb0_ref[...])
    h = jnp.dot(h, nw1_ref[...],
                preferred_element_type=jnp.float32) + nb1_ref[...]
    out_ref[0] = h


def kernel(x, fe_W0, fe_b0, fe_W1, fe_b1, fn_W0, fn_b0, fn_W1, fn_b1):
    B, N, D = x.shape
    F1 = fe_W0.shape[1]
    F2 = fe_W1.shape[1]
    FO = fn_W1.shape[1]

    w0a = fe_W0[:D]
    w0b = fe_W0[D:2 * D]
    w0d = fe_W0[2 * D:]

    full = lambda shape: pl.BlockSpec(shape, lambda b: (0,) * len(shape))

    return pl.pallas_call(
        _mp_kernel,
        grid=(B,),
        in_specs=[
            pl.BlockSpec((1, N, D), lambda b: (b, 0, 0)),
            full((D, F1)), full((D, F1)), full((1, F1)), full((1, F1)),
            full((F1, F2)), full((1, F2)),
            full((F2 + D, fn_W0.shape[1])), full((1, fn_W0.shape[1])),
            full((fn_W0.shape[1], FO)), full((1, FO)),
        ],
        out_specs=pl.BlockSpec((1, N, FO), lambda b: (b, 0, 0)),
        out_shape=jax.ShapeDtypeStruct((B, N, FO), jnp.float32),
        compiler_params=pltpu.CompilerParams(
            dimension_semantics=("parallel",)),
    )(x, w0a, w0b, w0d, fe_b0.reshape(1, -1), fe_W1, fe_b1.reshape(1, -1),
      fn_W0, fn_b0.reshape(1, -1), fn_W1, fn_b1.reshape(1, -1))


# speedup vs baseline: 1.4175x; 1.4175x over previous
import jax
import jax.numpy as jnp
from jax.experimental import pallas as pl
from jax.experimental.pallas import tpu as pltpu

_ALPHA = 0.2
_PACK = 4


def _lrelu(v):
    return jnp.maximum(v, _ALPHA * v)


def _mp_kernel(x_ref, x4_ref, w0a4_ref, w0b4_ref, p4_ref, b04_ref,
               w1bd_ref, b14_ref,
               nw0_ref, nb0_ref, nw1_ref, nb1_ref, out_ref):
    x = x_ref[0]
    n, d = x.shape
    k = _PACK
    m = n // k

    u4 = jnp.dot(x4_ref[0], w0a4_ref[...],
                 preferred_element_type=jnp.float32)
    v4 = jnp.dot(x, w0b4_ref[...], preferred_element_type=jnp.float32)
    v4 = v4 + b04_ref[...]

    xx = x * x
    g = jax.lax.dot_general(x, x, (((1,), (1,)), ((), ())),
                            preferred_element_type=jnp.float32)
    sq_col = jnp.sum(xx, axis=1, keepdims=True)
    ones_row = jnp.ones((1, d), jnp.float32)
    sq_row = jax.lax.dot_general(ones_row, xx, (((1,), (1,)), ((), ())),
                                 preferred_element_type=jnp.float32)
    dist = jnp.sqrt(jnp.maximum(sq_col + sq_row - 2.0 * g, 0.0))

    dist4 = jnp.transpose(dist.reshape(m, k, n), (0, 2, 1)).reshape(m * n, k)
    dterm = jnp.dot(dist4, p4_ref[...],
                    preferred_element_type=jnp.float32)

    e = u4[:, None, :] + v4[None, :, :] + dterm.reshape(m, n, k * d)
    e = _lrelu(e)

    e2 = jnp.dot(e.reshape(m * n, k * d), w1bd_ref[...],
                 preferred_element_type=jnp.float32) + b14_ref[...]
    e2 = _lrelu(e2)

    agg4 = jnp.sum(e2.reshape(m, n, -1), axis=1)
    f2 = agg4.shape[1] // k
    agg = jnp.stack([agg4[:, i * f2:(i + 1) * f2] for i in range(k)],
                    axis=1).reshape(n, f2)

    h = jnp.concatenate([agg, x], axis=1)
    h = _lrelu(jnp.dot(h, nw0_ref[...],
                       preferred_element_type=jnp.float32) + nb0_ref[...])
    h = jnp.dot(h, nw1_ref[...],
                preferred_element_type=jnp.float32) + nb1_ref[...]
    out_ref[0] = h


def kernel(x, fe_W0, fe_b0, fe_W1, fe_b1, fn_W0, fn_b0, fn_W1, fn_b1):
    B, N, D = x.shape
    F1 = fe_W0.shape[1]
    F2 = fe_W1.shape[1]
    FO = fn_W1.shape[1]
    k = _PACK

    w0a = fe_W0[:D]
    w0b = fe_W0[D:2 * D]
    w0d = fe_W0[2 * D]

    w0a4 = jnp.zeros((k * D, k * F1), jnp.float32)
    w1bd = jnp.zeros((k * F1, k * F2), jnp.float32)
    p4 = jnp.zeros((k, k * F1), jnp.float32)
    for i in range(k):
        w0a4 = w0a4.at[i * D:(i + 1) * D, i * F1:(i + 1) * F1].set(w0a)
        w1bd = w1bd.at[i * F1:(i + 1) * F1, i * F2:(i + 1) * F2].set(fe_W1)
        p4 = p4.at[i, i * F1:(i + 1) * F1].set(w0d)
    w0b4 = jnp.tile(w0b, (1, k))
    b04 = jnp.tile(fe_b0.reshape(1, -1), (1, k))
    b14 = jnp.tile(fe_b1.reshape(1, -1), (1, k))

    full = lambda shape: pl.BlockSpec(shape, lambda b: (0,) * len(shape))

    return pl.pallas_call(
        _mp_kernel,
        grid=(B,),
        in_specs=[
            pl.BlockSpec((1, N, D), lambda b: (b, 0, 0)),
            pl.BlockSpec((1, N // k, k * D), lambda b: (b, 0, 0)),
            full(w0a4.shape), full(w0b4.shape), full(p4.shape), full(b04.shape),
            full(w1bd.shape), full(b14.shape),
            full(fn_W0.shape), full((1, fn_W0.shape[1])),
            full(fn_W1.shape), full((1, FO)),
        ],
        out_specs=pl.BlockSpec((1, N, FO), lambda b: (b, 0, 0)),
        out_shape=jax.ShapeDtypeStruct((B, N, FO), jnp.float32),
        compiler_params=pltpu.CompilerParams(
            dimension_semantics=("parallel",)),
    )(x, x.reshape(B, N // k, k * D), w0a4, w0b4, p4, b04, w1bd, b14,
      fn_W0, fn_b0.reshape(1, -1), fn_W1, fn_b1.reshape(1, -1))


# speedup vs baseline: 1.4629x; 1.0320x over previous
import jax
import jax.numpy as jnp
from jax.experimental import pallas as pl
from jax.experimental.pallas import tpu as pltpu

_ALPHA = 0.2
_PACK = 4


def _lrelu(v):
    return jnp.maximum(v, _ALPHA * v)


def _edge_kernel(x_ref, x4_ref, w0a4_ref, w0b4_ref, p4_ref, b04_ref,
                 w1bd_ref, b14_ref, agg_ref):
    x = x_ref[0]
    n, d = x.shape
    k = _PACK
    m = n // k

    u4 = jnp.dot(x4_ref[0], w0a4_ref[...],
                 preferred_element_type=jnp.float32)
    v4 = jnp.dot(x, w0b4_ref[...], preferred_element_type=jnp.float32)
    v4 = v4 + b04_ref[...]

    xx = x * x
    g = jax.lax.dot_general(x, x, (((1,), (1,)), ((), ())),
                            preferred_element_type=jnp.float32)
    sq_col = jnp.sum(xx, axis=1, keepdims=True)
    ones_row = jnp.ones((1, d), jnp.float32)
    sq_row = jax.lax.dot_general(ones_row, xx, (((1,), (1,)), ((), ())),
                                 preferred_element_type=jnp.float32)
    dist = jnp.sqrt(jnp.maximum(sq_col + sq_row - 2.0 * g, 0.0))

    dist4 = jnp.transpose(dist.reshape(m, k, n), (0, 2, 1)).reshape(m * n, k)
    dterm = jnp.dot(dist4, p4_ref[...],
                    preferred_element_type=jnp.float32)

    e = u4[:, None, :] + v4[None, :, :] + dterm.reshape(m, n, k * d)
    e = _lrelu(e)

    e2 = jnp.dot(e.reshape(m * n, k * d), w1bd_ref[...],
                 preferred_element_type=jnp.float32) + b14_ref[...]
    e2 = _lrelu(e2)

    agg_ref[0] = jnp.sum(e2.reshape(m, n, -1), axis=1)


def _node_kernel(aggp_ref, xp_ref, na2_ref, nb2_ref, nb02_ref,
                 nw1bd_ref, nb12_ref, out_ref):
    h = jnp.dot(aggp_ref[...], na2_ref[...],
                preferred_element_type=jnp.float32)
    h = h + jnp.dot(xp_ref[...], nb2_ref[...],
                    preferred_element_type=jnp.float32)
    h = _lrelu(h + nb02_ref[...])
    out_ref[...] = jnp.dot(h, nw1bd_ref[...],
                           preferred_element_type=jnp.float32) + nb12_ref[...]


def _blockdiag(w, k):
    fi, fo = w.shape
    out = jnp.zeros((k * fi, k * fo), jnp.float32)
    for i in range(k):
        out = out.at[i * fi:(i + 1) * fi, i * fo:(i + 1) * fo].set(w)
    return out


def kernel(x, fe_W0, fe_b0, fe_W1, fe_b1, fn_W0, fn_b0, fn_W1, fn_b1):
    B, N, D = x.shape
    F1 = fe_W0.shape[1]
    F2 = fe_W1.shape[1]
    FN = fn_W0.shape[1]
    FO = fn_W1.shape[1]
    k = _PACK
    m = N // k

    w0a = fe_W0[:D]
    w0b = fe_W0[D:2 * D]
    w0d = fe_W0[2 * D]

    w0a4 = _blockdiag(w0a, k)
    w1bd = _blockdiag(fe_W1, k)
    p4 = jnp.zeros((k, k * F1), jnp.float32)
    for i in range(k):
        p4 = p4.at[i, i * F1:(i + 1) * F1].set(w0d)
    w0b4 = jnp.tile(w0b, (1, k))
    b04 = jnp.tile(fe_b0.reshape(1, -1), (1, k))
    b14 = jnp.tile(fe_b1.reshape(1, -1), (1, k))

    full = lambda shape: pl.BlockSpec(shape, lambda b: (0,) * len(shape))

    agg4 = pl.pallas_call(
        _edge_kernel,
        grid=(B,),
        in_specs=[
            pl.BlockSpec((1, N, D), lambda b: (b, 0, 0)),
            pl.BlockSpec((1, m, k * D), lambda b: (b, 0, 0)),
            full(w0a4.shape), full(w0b4.shape), full(p4.shape), full(b04.shape),
            full(w1bd.shape), full(b14.shape),
        ],
        out_specs=pl.BlockSpec((1, m, k * F2), lambda b: (b, 0, 0)),
        out_shape=jax.ShapeDtypeStruct((B, m, k * F2), jnp.float32),
        compiler_params=pltpu.CompilerParams(
            dimension_semantics=("parallel",)),
    )(x, x.reshape(B, m, k * D), w0a4, w0b4, p4, b04, w1bd, b14)

    aggp = agg4.reshape(B * N // 2, 2 * F2)
    xp = x.reshape(B * N // 2, 2 * D)
    na2 = _blockdiag(fn_W0[:F2], 2)
    nb2 = _blockdiag(fn_W0[F2:], 2)
    nb02 = jnp.tile(fn_b0.reshape(1, -1), (1, 2))
    nw1bd = _blockdiag(fn_W1, 2)
    nb12 = jnp.tile(fn_b1.reshape(1, -1), (1, 2))

    outp = pl.pallas_call(
        _node_kernel,
        out_shape=jax.ShapeDtypeStruct((B * N // 2, 2 * FO), jnp.float32),
    )(aggp, xp, na2, nb2, nb02, nw1bd, nb12)

    return outp.reshape(B, N, FO)


# speedup vs baseline: 1.5773x; 1.0782x over previous
import jax
import jax.numpy as jnp
from jax.experimental import pallas as pl
from jax.experimental.pallas import tpu as pltpu

_ALPHA = 0.2
_PACK = 4
_BB = 2


def _lrelu(v):
    return jnp.maximum(v, _ALPHA * v)


def _edge_kernel(x_ref, x4_ref, w0a4_ref, w0b4_ref, p4_ref, b04_ref,
                 w1bd_ref, b14_ref, agg_ref):
    k = _PACK
    for s in range(x_ref.shape[0]):
        x = x_ref[s]
        n, d = x.shape
        m = n // k

        u4 = jnp.dot(x4_ref[s], w0a4_ref[...],
                     preferred_element_type=jnp.float32)
        v4 = jnp.dot(x, w0b4_ref[...], preferred_element_type=jnp.float32)
        v4 = v4 + b04_ref[...]

        xx = x * x
        g = jax.lax.dot_general(x, x, (((1,), (1,)), ((), ())),
                                preferred_element_type=jnp.float32)
        sq_col = jnp.sum(xx, axis=1, keepdims=True)
        ones_row = jnp.ones((1, d), jnp.float32)
        sq_row = jax.lax.dot_general(ones_row, xx, (((1,), (1,)), ((), ())),
                                     preferred_element_type=jnp.float32)
        dist = jnp.sqrt(jnp.maximum(sq_col + sq_row - 2.0 * g, 0.0))

        dist4 = jnp.transpose(dist.reshape(m, k, n),
                              (0, 2, 1)).reshape(m * n, k)
        dterm = jnp.dot(dist4, p4_ref[...],
                        preferred_element_type=jnp.float32)

        e = u4[:, None, :] + v4[None, :, :] + dterm.reshape(m, n, k * d)
        e = _lrelu(e)

        e2 = jnp.dot(e.reshape(m * n, k * d), w1bd_ref[...],
                     preferred_element_type=jnp.float32) + b14_ref[...]
        e2 = _lrelu(e2)

        agg_ref[s] = jnp.sum(e2.reshape(m, n, -1), axis=1)


def _node_kernel(aggp_ref, xp_ref, na2_ref, nb2_ref, nb02_ref,
                 nw1bd_ref, nb12_ref, out_ref):
    h = jnp.dot(aggp_ref[...], na2_ref[...],
                preferred_element_type=jnp.float32)
    h = h + jnp.dot(xp_ref[...], nb2_ref[...],
                    preferred_element_type=jnp.float32)
    h = _lrelu(h + nb02_ref[...])
    out_ref[...] = jnp.dot(h, nw1bd_ref[...],
                           preferred_element_type=jnp.float32) + nb12_ref[...]


def _blockdiag(w, k):
    fi, fo = w.shape
    out = jnp.zeros((k * fi, k * fo), jnp.float32)
    for i in range(k):
        out = out.at[i * fi:(i + 1) * fi, i * fo:(i + 1) * fo].set(w)
    return out


def kernel(x, fe_W0, fe_b0, fe_W1, fe_b1, fn_W0, fn_b0, fn_W1, fn_b1):
    B, N, D = x.shape
    F1 = fe_W0.shape[1]
    F2 = fe_W1.shape[1]
    FN = fn_W0.shape[1]
    FO = fn_W1.shape[1]
    k = _PACK
    m = N // k

    w0a = fe_W0[:D]
    w0b = fe_W0[D:2 * D]
    w0d = fe_W0[2 * D]

    w0a4 = _blockdiag(w0a, k)
    w1bd = _blockdiag(fe_W1, k)
    p4 = jnp.zeros((k, k * F1), jnp.float32)
    for i in range(k):
        p4 = p4.at[i, i * F1:(i + 1) * F1].set(w0d)
    w0b4 = jnp.tile(w0b, (1, k))
    b04 = jnp.tile(fe_b0.reshape(1, -1), (1, k))
    b14 = jnp.tile(fe_b1.reshape(1, -1), (1, k))

    full = lambda shape: pl.BlockSpec(shape, lambda b: (0,) * len(shape))

    bb = _BB
    agg4 = pl.pallas_call(
        _edge_kernel,
        grid=(B // bb,),
        in_specs=[
            pl.BlockSpec((bb, N, D), lambda b: (b, 0, 0)),
            pl.BlockSpec((bb, m, k * D), lambda b: (b, 0, 0)),
            full(w0a4.shape), full(w0b4.shape), full(p4.shape), full(b04.shape),
            full(w1bd.shape), full(b14.shape),
        ],
        out_specs=pl.BlockSpec((bb, m, k * F2), lambda b: (b, 0, 0)),
        out_shape=jax.ShapeDtypeStruct((B, m, k * F2), jnp.float32),
        compiler_params=pltpu.CompilerParams(
            dimension_semantics=("parallel",)),
    )(x, x.reshape(B, m, k * D), w0a4, w0b4, p4, b04, w1bd, b14)

    aggp = agg4.reshape(B * N // 2, 2 * F2)
    xp = x.reshape(B * N // 2, 2 * D)
    na2 = _blockdiag(fn_W0[:F2], 2)
    nb2 = _blockdiag(fn_W0[F2:], 2)
    nb02 = jnp.tile(fn_b0.reshape(1, -1), (1, 2))
    nw1bd = _blockdiag(fn_W1, 2)
    nb12 = jnp.tile(fn_b1.reshape(1, -1), (1, 2))

    outp = pl.pallas_call(
        _node_kernel,
        out_shape=jax.ShapeDtypeStruct((B * N // 2, 2 * FO), jnp.float32),
    )(aggp, xp, na2, nb2, nb02, nw1bd, nb12)

    return outp.reshape(B, N, FO)


# speedup vs baseline: 1.6561x; 1.0499x over previous
import jax
import jax.numpy as jnp
from jax.experimental import pallas as pl
from jax.experimental.pallas import tpu as pltpu

_ALPHA = 0.2
_PACK = 4
_BB = 4


def _lrelu(v):
    return jnp.maximum(v, _ALPHA * v)


def _edge_kernel(x_ref, x4_ref, w0a4_ref, w0b4_ref, p4_ref, b04_ref,
                 w1bd_ref, b14_ref, agg_ref):
    k = _PACK
    for s in range(x_ref.shape[0]):
        x = x_ref[s]
        n, d = x.shape
        m = n // k

        u4 = jnp.dot(x4_ref[s], w0a4_ref[...],
                     preferred_element_type=jnp.float32)
        v4 = jnp.dot(x, w0b4_ref[...], preferred_element_type=jnp.float32)
        v4 = v4 + b04_ref[...]

        xx = x * x
        g = jax.lax.dot_general(x, x, (((1,), (1,)), ((), ())),
                                preferred_element_type=jnp.float32)
        sq_col = jnp.sum(xx, axis=1, keepdims=True)
        ones_row = jnp.ones((1, d), jnp.float32)
        sq_row = jax.lax.dot_general(ones_row, xx, (((1,), (1,)), ((), ())),
                                     preferred_element_type=jnp.float32)
        dist = jnp.sqrt(jnp.maximum(sq_col + sq_row - 2.0 * g, 0.0))

        dist4 = jnp.transpose(dist.reshape(m, k, n),
                              (0, 2, 1)).reshape(m * n, k)
        dterm = jnp.dot(dist4, p4_ref[...],
                        preferred_element_type=jnp.float32)

        e = u4[:, None, :] + v4[None, :, :] + dterm.reshape(m, n, k * d)
        e = _lrelu(e)

        e2 = jnp.dot(e.reshape(m * n, k * d), w1bd_ref[...],
                     preferred_element_type=jnp.float32) + b14_ref[...]
        e2 = _lrelu(e2)

        agg_ref[s] = jnp.sum(e2.reshape(m, n, -1), axis=1)


def _node_kernel(aggp_ref, xp_ref, na2_ref, nb2_ref, nb02_ref,
                 nw1bd_ref, nb12_ref, out_ref):
    h = jnp.dot(aggp_ref[...], na2_ref[...],
                preferred_element_type=jnp.float32)
    h = h + jnp.dot(xp_ref[...], nb2_ref[...],
                    preferred_element_type=jnp.float32)
    h = _lrelu(h + nb02_ref[...])
    out_ref[...] = jnp.dot(h, nw1bd_ref[...],
                           preferred_element_type=jnp.float32) + nb12_ref[...]


def _blockdiag(w, k):
    fi, fo = w.shape
    out = jnp.zeros((k * fi, k * fo), jnp.float32)
    for i in range(k):
        out = out.at[i * fi:(i + 1) * fi, i * fo:(i + 1) * fo].set(w)
    return out


def kernel(x, fe_W0, fe_b0, fe_W1, fe_b1, fn_W0, fn_b0, fn_W1, fn_b1):
    B, N, D = x.shape
    F1 = fe_W0.shape[1]
    F2 = fe_W1.shape[1]
    FN = fn_W0.shape[1]
    FO = fn_W1.shape[1]
    k = _PACK
    m = N // k

    w0a = fe_W0[:D]
    w0b = fe_W0[D:2 * D]
    w0d = fe_W0[2 * D]

    w0a4 = _blockdiag(w0a, k)
    w1bd = _blockdiag(fe_W1, k)
    p4 = jnp.zeros((k, k * F1), jnp.float32)
    for i in range(k):
        p4 = p4.at[i, i * F1:(i + 1) * F1].set(w0d)
    w0b4 = jnp.tile(w0b, (1, k))
    b04 = jnp.tile(fe_b0.reshape(1, -1), (1, k))
    b14 = jnp.tile(fe_b1.reshape(1, -1), (1, k))

    full = lambda shape: pl.BlockSpec(shape, lambda b: (0,) * len(shape))

    bb = _BB
    agg4 = pl.pallas_call(
        _edge_kernel,
        grid=(B // bb,),
        in_specs=[
            pl.BlockSpec((bb, N, D), lambda b: (b, 0, 0)),
            pl.BlockSpec((bb, m, k * D), lambda b: (b, 0, 0)),
            full(w0a4.shape), full(w0b4.shape), full(p4.shape), full(b04.shape),
            full(w1bd.shape), full(b14.shape),
        ],
        out_specs=pl.BlockSpec((bb, m, k * F2), lambda b: (b, 0, 0)),
        out_shape=jax.ShapeDtypeStruct((B, m, k * F2), jnp.float32),
        compiler_params=pltpu.CompilerParams(
            dimension_semantics=("parallel",)),
    )(x, x.reshape(B, m, k * D), w0a4, w0b4, p4, b04, w1bd, b14)

    aggp = agg4.reshape(B * N // 2, 2 * F2)
    xp = x.reshape(B * N // 2, 2 * D)
    na2 = _blockdiag(fn_W0[:F2], 2)
    nb2 = _blockdiag(fn_W0[F2:], 2)
    nb02 = jnp.tile(fn_b0.reshape(1, -1), (1, 2))
    nw1bd = _blockdiag(fn_W1, 2)
    nb12 = jnp.tile(fn_b1.reshape(1, -1), (1, 2))

    outp = pl.pallas_call(
        _node_kernel,
        out_shape=jax.ShapeDtypeStruct((B * N // 2, 2 * FO), jnp.float32),
    )(aggp, xp, na2, nb2, nb02, nw1bd, nb12)

    return outp.reshape(B, N, FO)


# speedup vs baseline: 1.7182x; 1.0375x over previous
import jax
import jax.numpy as jnp
from jax.experimental import pallas as pl
from jax.experimental.pallas import tpu as pltpu

_ALPHA = 0.2
_PACK = 4
_BB = 4


def _lrelu(v):
    return jnp.maximum(v, _ALPHA * v)


def _mp_kernel(x_ref, x4_ref, feW0_ref, feb0_ref, feW1_ref, feb1_ref,
               fnW0_ref, fnb0_ref, fnW1_ref, fnb1_ref, out_ref,
               w0a4_s, w1bd_s, p4_s, na4_s, nb4_s, nw14_s):
    k = _PACK
    n, d = x_ref.shape[1], x_ref.shape[2]
    m = n // k
    f1 = feW1_ref.shape[0]
    f2 = feW1_ref.shape[1]
    fn = fnW0_ref.shape[1]
    fo = fnW1_ref.shape[1]

    @pl.when(pl.program_id(0) == 0)
    def _prep():
        w0a4_s[...] = jnp.zeros_like(w0a4_s)
        w1bd_s[...] = jnp.zeros_like(w1bd_s)
        p4_s[...] = jnp.zeros_like(p4_s)
        na4_s[...] = jnp.zeros_like(na4_s)
        nb4_s[...] = jnp.zeros_like(nb4_s)
        nw14_s[...] = jnp.zeros_like(nw14_s)
        for i in range(k):
            w0a4_s[i * d:(i + 1) * d, i * f1:(i + 1) * f1] = feW0_ref[0:d]
            w1bd_s[i * f1:(i + 1) * f1, i * f2:(i + 1) * f2] = feW1_ref[...]
            p4_s[i:i + 1, i * f1:(i + 1) * f1] = feW0_ref[2 * d:2 * d + 1]
            na4_s[i * f2:(i + 1) * f2, i * fn:(i + 1) * fn] = fnW0_ref[0:f2]
            nb4_s[i * d:(i + 1) * d, i * fn:(i + 1) * fn] = fnW0_ref[f2:]
            nw14_s[i * fn:(i + 1) * fn, i * fo:(i + 1) * fo] = fnW1_ref[...]

    w0b = feW0_ref[d:2 * d]
    b04 = jnp.tile(feb0_ref[...], (1, k))
    b14 = jnp.tile(feb1_ref[...], (1, k))
    nb04 = jnp.tile(fnb0_ref[...], (1, k))
    nb14 = jnp.tile(fnb1_ref[...], (1, k))

    for s in range(x_ref.shape[0]):
        x = x_ref[s]

        u4 = jnp.dot(x4_ref[s], w0a4_s[...],
                     preferred_element_type=jnp.float32)
        v = jnp.dot(x, w0b, preferred_element_type=jnp.float32)
        v4 = jnp.tile(v, (1, k)) + b04

        xx = x * x
        g = jax.lax.dot_general(x, x, (((1,), (1,)), ((), ())),
                                preferred_element_type=jnp.float32)
        sq_col = jnp.sum(xx, axis=1, keepdims=True)
        ones_row = jnp.ones((1, d), jnp.float32)
        sq_row = jax.lax.dot_general(ones_row, xx, (((1,), (1,)), ((), ())),
                                     preferred_element_type=jnp.float32)
        dist = jnp.sqrt(jnp.maximum(sq_col + sq_row - 2.0 * g, 0.0))

        dist4 = jnp.transpose(dist.reshape(m, k, n),
                              (0, 2, 1)).reshape(m * n, k)
        dterm = jnp.dot(dist4, p4_s[...],
                        preferred_element_type=jnp.float32)

        e = u4[:, None, :] + v4[None, :, :] + dterm.reshape(m, n, k * d)
        e = _lrelu(e)

        e2 = jnp.dot(e.reshape(m * n, k * d), w1bd_s[...],
                     preferred_element_type=jnp.float32) + b14
        e2 = _lrelu(e2)

        agg4 = jnp.sum(e2.reshape(m, n, -1), axis=1)

        h = jnp.dot(agg4, na4_s[...], preferred_element_type=jnp.float32)
        h = h + jnp.dot(x4_ref[s], nb4_s[...],
                        preferred_element_type=jnp.float32)
        h = _lrelu(h + nb04)
        out_ref[s] = jnp.dot(h, nw14_s[...],
                             preferred_element_type=jnp.float32) + nb14


def kernel(x, fe_W0, fe_b0, fe_W1, fe_b1, fn_W0, fn_b0, fn_W1, fn_b1):
    B, N, D = x.shape
    F1 = fe_W0.shape[1]
    F2 = fe_W1.shape[1]
    FN = fn_W0.shape[1]
    FO = fn_W1.shape[1]
    k = _PACK
    m = N // k
    bb = _BB

    full = lambda shape: pl.BlockSpec(shape, lambda b: (0,) * len(shape))

    out4 = pl.pallas_call(
        _mp_kernel,
        grid=(B // bb,),
        in_specs=[
            pl.BlockSpec((bb, N, D), lambda b: (b, 0, 0)),
            pl.BlockSpec((bb, m, k * D), lambda b: (b, 0, 0)),
            full(fe_W0.shape), full((1, F1)), full(fe_W1.shape), full((1, F2)),
            full(fn_W0.shape), full((1, FN)), full(fn_W1.shape), full((1, FO)),
        ],
        out_specs=pl.BlockSpec((bb, m, k * FO), lambda b: (b, 0, 0)),
        out_shape=jax.ShapeDtypeStruct((B, m, k * FO), jnp.float32),
        scratch_shapes=[
            pltpu.VMEM((k * D, k * F1), jnp.float32),
            pltpu.VMEM((k * F1, k * F2), jnp.float32),
            pltpu.VMEM((k, k * F1), jnp.float32),
            pltpu.VMEM((k * F2, k * FN), jnp.float32),
            pltpu.VMEM((k * D, k * FN), jnp.float32),
            pltpu.VMEM((k * FN, k * FO), jnp.float32),
        ],
    )(x, x.reshape(B, m, k * D), fe_W0, fe_b0.reshape(1, -1), fe_W1,
      fe_b1.reshape(1, -1), fn_W0, fn_b0.reshape(1, -1), fn_W1,
      fn_b1.reshape(1, -1))

    return out4.reshape(B, N, FO)


# speedup vs baseline: 1.7781x; 1.0349x over previous
import jax
import jax.numpy as jnp
from jax.experimental import pallas as pl
from jax.experimental.pallas import tpu as pltpu

_ALPHA = 0.2
_PACK = 4
_BB = 8


def _lrelu(v):
    return jnp.maximum(v, _ALPHA * v)


def _mp_kernel(x_ref, x4_ref, feW0_ref, feb0_ref, feW1_ref, feb1_ref,
               fnW0_ref, fnb0_ref, fnW1_ref, fnb1_ref, out_ref,
               w0a4_s, w1bd_s, p4_s, na4_s, nb4_s, nw14_s):
    k = _PACK
    n, d = x_ref.shape[1], x_ref.shape[2]
    m = n // k
    f1 = feW1_ref.shape[0]
    f2 = feW1_ref.shape[1]
    fn = fnW0_ref.shape[1]
    fo = fnW1_ref.shape[1]

    @pl.when(pl.program_id(0) == 0)
    def _prep():
        w0a4_s[...] = jnp.zeros_like(w0a4_s)
        w1bd_s[...] = jnp.zeros_like(w1bd_s)
        p4_s[...] = jnp.zeros_like(p4_s)
        na4_s[...] = jnp.zeros_like(na4_s)
        nb4_s[...] = jnp.zeros_like(nb4_s)
        nw14_s[...] = jnp.zeros_like(nw14_s)
        for i in range(k):
            w0a4_s[i * d:(i + 1) * d, i * f1:(i + 1) * f1] = feW0_ref[0:d]
            w1bd_s[i * f1:(i + 1) * f1, i * f2:(i + 1) * f2] = feW1_ref[...]
            p4_s[i:i + 1, i * f1:(i + 1) * f1] = feW0_ref[2 * d:2 * d + 1]
            na4_s[i * f2:(i + 1) * f2, i * fn:(i + 1) * fn] = fnW0_ref[0:f2]
            nb4_s[i * d:(i + 1) * d, i * fn:(i + 1) * fn] = fnW0_ref[f2:]
            nw14_s[i * fn:(i + 1) * fn, i * fo:(i + 1) * fo] = fnW1_ref[...]

    w0b = feW0_ref[d:2 * d]
    b04 = jnp.tile(feb0_ref[...], (1, k))
    b14 = jnp.tile(feb1_ref[...], (1, k))
    nb04 = jnp.tile(fnb0_ref[...], (1, k))
    nb14 = jnp.tile(fnb1_ref[...], (1, k))

    for s in range(x_ref.shape[0]):
        x = x_ref[s]

        u4 = jnp.dot(x4_ref[s], w0a4_s[...],
                     preferred_element_type=jnp.float32)
        v = jnp.dot(x, w0b, preferred_element_type=jnp.float32)
        v4 = jnp.tile(v, (1, k)) + b04

        xx = x * x
        g = jax.lax.dot_general(x, x, (((1,), (1,)), ((), ())),
                                preferred_element_type=jnp.float32)
        sq_col = jnp.sum(xx, axis=1, keepdims=True)
        ones_row = jnp.ones((1, d), jnp.float32)
        sq_row = jax.lax.dot_general(ones_row, xx, (((1,), (1,)), ((), ())),
                                     preferred_element_type=jnp.float32)
        dist = jnp.sqrt(jnp.maximum(sq_col + sq_row - 2.0 * g, 0.0))

        dist4 = jnp.transpose(dist.reshape(m, k, n),
                              (0, 2, 1)).reshape(m * n, k)
        dterm = jnp.dot(dist4, p4_s[...],
                        preferred_element_type=jnp.float32)

        e = u4[:, None, :] + v4[None, :, :] + dterm.reshape(m, n, k * d)
        e = _lrelu(e)

        e2 = jnp.dot(e.reshape(m * n, k * d), w1bd_s[...],
                     preferred_element_type=jnp.float32) + b14
        e2 = _lrelu(e2)

        agg4 = jnp.sum(e2.reshape(m, n, -1), axis=1)

        h = jnp.dot(agg4, na4_s[...], preferred_element_type=jnp.float32)
        h = h + jnp.dot(x4_ref[s], nb4_s[...],
                        preferred_element_type=jnp.float32)
        h = _lrelu(h + nb04)
        out_ref[s] = jnp.dot(h, nw14_s[...],
                             preferred_element_type=jnp.float32) + nb14


def kernel(x, fe_W0, fe_b0, fe_W1, fe_b1, fn_W0, fn_b0, fn_W1, fn_b1):
    B, N, D = x.shape
    F1 = fe_W0.shape[1]
    F2 = fe_W1.shape[1]
    FN = fn_W0.shape[1]
    FO = fn_W1.shape[1]
    k = _PACK
    m = N // k
    bb = _BB

    full = lambda shape: pl.BlockSpec(shape, lambda b: (0,) * len(shape))

    out4 = pl.pallas_call(
        _mp_kernel,
        grid=(B // bb,),
        in_specs=[
            pl.BlockSpec((bb, N, D), lambda b: (b, 0, 0)),
            pl.BlockSpec((bb, m, k * D), lambda b: (b, 0, 0)),
            full(fe_W0.shape), full((1, F1)), full(fe_W1.shape), full((1, F2)),
            full(fn_W0.shape), full((1, FN)), full(fn_W1.shape), full((1, FO)),
        ],
        out_specs=pl.BlockSpec((bb, m, k * FO), lambda b: (b, 0, 0)),
        out_shape=jax.ShapeDtypeStruct((B, m, k * FO), jnp.float32),
        scratch_shapes=[
            pltpu.VMEM((k * D, k * F1), jnp.float32),
            pltpu.VMEM((k * F1, k * F2), jnp.float32),
            pltpu.VMEM((k, k * F1), jnp.float32),
            pltpu.VMEM((k * F2, k * FN), jnp.float32),
            pltpu.VMEM((k * D, k * FN), jnp.float32),
            pltpu.VMEM((k * FN, k * FO), jnp.float32),
        ],
    )(x, x.reshape(B, m, k * D), fe_W0, fe_b0.reshape(1, -1), fe_W1,
      fe_b1.reshape(1, -1), fn_W0, fn_b0.reshape(1, -1), fn_W1,
      fn_b1.reshape(1, -1))

    return out4.reshape(B, N, FO)


# speedup vs baseline: 1.8125x; 1.0193x over previous
import jax
import jax.numpy as jnp
from jax.experimental import pallas as pl
from jax.experimental.pallas import tpu as pltpu

_ALPHA = 0.2
_PACK = 4
_BB = 16


def _lrelu(v):
    return jnp.maximum(v, _ALPHA * v)


def _mp_kernel(x_ref, x4_ref, feW0_ref, feb0_ref, feW1_ref, feb1_ref,
               fnW0_ref, fnb0_ref, fnW1_ref, fnb1_ref, out_ref,
               w0a4_s, w1bd_s, p4_s, na4_s, nb4_s, nw14_s):
    k = _PACK
    n, d = x_ref.shape[1], x_ref.shape[2]
    m = n // k
    f1 = feW1_ref.shape[0]
    f2 = feW1_ref.shape[1]
    fn = fnW0_ref.shape[1]
    fo = fnW1_ref.shape[1]

    @pl.when(pl.program_id(0) == 0)
    def _prep():
        w0a4_s[...] = jnp.zeros_like(w0a4_s)
        w1bd_s[...] = jnp.zeros_like(w1bd_s)
        p4_s[...] = jnp.zeros_like(p4_s)
        na4_s[...] = jnp.zeros_like(na4_s)
        nb4_s[...] = jnp.zeros_like(nb4_s)
        nw14_s[...] = jnp.zeros_like(nw14_s)
        for i in range(k):
            w0a4_s[i * d:(i + 1) * d, i * f1:(i + 1) * f1] = feW0_ref[0:d]
            w1bd_s[i * f1:(i + 1) * f1, i * f2:(i + 1) * f2] = feW1_ref[...]
            p4_s[i:i + 1, i * f1:(i + 1) * f1] = feW0_ref[2 * d:2 * d + 1]
            na4_s[i * f2:(i + 1) * f2, i * fn:(i + 1) * fn] = fnW0_ref[0:f2]
            nb4_s[i * d:(i + 1) * d, i * fn:(i + 1) * fn] = fnW0_ref[f2:]
            nw14_s[i * fn:(i + 1) * fn, i * fo:(i + 1) * fo] = fnW1_ref[...]

    w0b = feW0_ref[d:2 * d]
    b04 = jnp.tile(feb0_ref[...], (1, k))
    b14 = jnp.tile(feb1_ref[...], (1, k))
    nb04 = jnp.tile(fnb0_ref[...], (1, k))
    nb14 = jnp.tile(fnb1_ref[...], (1, k))

    for s in range(x_ref.shape[0]):
        x = x_ref[s]

        u4 = jnp.dot(x4_ref[s], w0a4_s[...],
                     preferred_element_type=jnp.float32)
        v = jnp.dot(x, w0b, preferred_element_type=jnp.float32)
        v4 = jnp.tile(v, (1, k)) + b04

        xx = x * x
        g = jax.lax.dot_general(x, x, (((1,), (1,)), ((), ())),
                                preferred_element_type=jnp.float32)
        sq_col = jnp.sum(xx, axis=1, keepdims=True)
        ones_row = jnp.ones((1, d), jnp.float32)
        sq_row = jax.lax.dot_general(ones_row, xx, (((1,), (1,)), ((), ())),
                                     preferred_element_type=jnp.float32)
        dist = jnp.sqrt(jnp.maximum(sq_col + sq_row - 2.0 * g, 0.0))

        dist4 = jnp.transpose(dist.reshape(m, k, n),
                              (0, 2, 1)).reshape(m * n, k)
        dterm = jnp.dot(dist4, p4_s[...],
                        preferred_element_type=jnp.float32)

        e = u4[:, None, :] + v4[None, :, :] + dterm.reshape(m, n, k * d)
        e = _lrelu(e)

        e2 = jnp.dot(e.reshape(m * n, k * d), w1bd_s[...],
                     preferred_element_type=jnp.float32) + b14
        e2 = _lrelu(e2)

        agg4 = jnp.sum(e2.reshape(m, n, -1), axis=1)

        h = jnp.dot(agg4, na4_s[...], preferred_element_type=jnp.float32)
        h = h + jnp.dot(x4_ref[s], nb4_s[...],
                        preferred_element_type=jnp.float32)
        h = _lrelu(h + nb04)
        out_ref[s] = jnp.dot(h, nw14_s[...],
                             preferred_element_type=jnp.float32) + nb14


def kernel(x, fe_W0, fe_b0, fe_W1, fe_b1, fn_W0, fn_b0, fn_W1, fn_b1):
    B, N, D = x.shape
    F1 = fe_W0.shape[1]
    F2 = fe_W1.shape[1]
    FN = fn_W0.shape[1]
    FO = fn_W1.shape[1]
    k = _PACK
    m = N // k
    bb = _BB

    full = lambda shape: pl.BlockSpec(shape, lambda b: (0,) * len(shape))

    out4 = pl.pallas_call(
        _mp_kernel,
        grid=(B // bb,),
        in_specs=[
            pl.BlockSpec((bb, N, D), lambda b: (b, 0, 0)),
            pl.BlockSpec((bb, m, k * D), lambda b: (b, 0, 0)),
            full(fe_W0.shape), full((1, F1)), full(fe_W1.shape), full((1, F2)),
            full(fn_W0.shape), full((1, FN)), full(fn_W1.shape), full((1, FO)),
        ],
        out_specs=pl.BlockSpec((bb, m, k * FO), lambda b: (b, 0, 0)),
        out_shape=jax.ShapeDtypeStruct((B, m, k * FO), jnp.float32),
        scratch_shapes=[
            pltpu.VMEM((k * D, k * F1), jnp.float32),
            pltpu.VMEM((k * F1, k * F2), jnp.float32),
            pltpu.VMEM((k, k * F1), jnp.float32),
            pltpu.VMEM((k * F2, k * FN), jnp.float32),
            pltpu.VMEM((k * D, k * FN), jnp.float32),
            pltpu.VMEM((k * FN, k * FO), jnp.float32),
        ],
    )(x, x.reshape(B, m, k * D), fe_W0, fe_b0.reshape(1, -1), fe_W1,
      fe_b1.reshape(1, -1), fn_W0, fn_b0.reshape(1, -1), fn_W1,
      fn_b1.reshape(1, -1))

    return out4.reshape(B, N, FO)
